# baseline (speedup 1.0000x reference)
import functools

import jax
import jax.numpy as jnp
from jax import lax
from jax.experimental import pallas as pl
from jax.experimental.pallas import tpu as pltpu
from jax.experimental.pallas import tpu_sc as plsc

_FACT = 128
_HIDDEN = 2048
_CHUNK = 128
_N_WORKERS = 32


def _sc_gather(table, ids, n_chunks):
    info = plsc.get_sparse_core_info()
    nc = info.num_cores
    b_per_w = n_chunks * _CHUNK
    total = _N_WORKERS * b_per_w
    seq = ids.shape[1]
    per_row = seq // b_per_w
    mesh = plsc.VectorSubcoreMesh(core_axis_name="c", subcore_axis_name="s")

    @functools.partial(
        pl.kernel,
        mesh=mesh,
        out_type=jax.ShapeDtypeStruct((total, _FACT), jnp.float32),
        scratch_types=[
            pltpu.VMEM((b_per_w,), jnp.int32),
            pltpu.VMEM((b_per_w, _FACT), jnp.float32),
            pltpu.SemaphoreType.DMA,
            pltpu.SemaphoreType.DMA,
        ],
    )
    def gather_kernel(table_hbm, ids_hbm, out_hbm, idx_v, rows_v, gsem, osem):
        wid = lax.axis_index("s") * nc + lax.axis_index("c")
        row = wid // per_row
        col0 = (wid % per_row) * b_per_w
        pltpu.sync_copy(ids_hbm.at[row, pl.ds(col0, b_per_w)], idx_v)
        gathers = []
        for j in range(n_chunks):
            gathers.append(
                pltpu.async_copy(
                    table_hbm.at[idx_v.at[pl.ds(j * _CHUNK, _CHUNK)]],
                    rows_v.at[pl.ds(j * _CHUNK, _CHUNK)],
                    gsem,
                )
            )
        base = wid * b_per_w
        outs = []
        for j in range(n_chunks):
            gathers[j].wait()
            outs.append(
                pltpu.async_copy(
                    rows_v.at[pl.ds(j * _CHUNK, _CHUNK)],
                    out_hbm.at[pl.ds(base + j * _CHUNK, _CHUNK)],
                    osem,
                )
            )
        for o in outs:
            o.wait()

    return gather_kernel(table, ids)


def _tc_project(x, w, m_blk):
    m = x.shape[0]

    def mm(x_ref, w_ref, o_ref):
        o_ref[...] = lax.dot_general(
            x_ref[...].astype(jnp.bfloat16),
            w_ref[...],
            (((1,), (1,)), ((), ())),
            preferred_element_type=jnp.float32,
        )

    return pl.pallas_call(
        mm,
        grid=(m // m_blk,),
        in_specs=[
            pl.BlockSpec((m_blk, _FACT), lambda i: (i, 0)),
            pl.BlockSpec((_HIDDEN, _FACT), lambda i: (0, 0)),
        ],
        out_specs=pl.BlockSpec((m_blk, _HIDDEN), lambda i: (i, 0)),
        out_shape=jax.ShapeDtypeStruct((m, _HIDDEN), jnp.float32),
        compiler_params=pltpu.CompilerParams(
            dimension_semantics=("parallel",),
        ),
    )(x, w.astype(jnp.bfloat16))


def kernel(input_ids, embed_weight, proj_weight):
    b, s = input_ids.shape
    total = b * s
    n_chunks = total // (_N_WORKERS * _CHUNK)
    x = _sc_gather(embed_weight, input_ids.astype(jnp.int32), n_chunks)
    y = _tc_project(x, proj_weight, m_blk=1024)
    return y.reshape(b, s, _HIDDEN)


# speedup vs baseline: 1.0029x; 1.0029x over previous
import functools

import jax
import jax.numpy as jnp
from jax import lax
from jax.experimental import pallas as pl
from jax.experimental.pallas import tpu as pltpu
from jax.experimental.pallas import tpu_sc as plsc

_FACT = 128
_HIDDEN = 2048
_CHUNK = 64
_N_WORKERS = 32


def _sc_gather(table, ids, n_chunks):
    info = plsc.get_sparse_core_info()
    nc = info.num_cores
    b_per_w = n_chunks * _CHUNK
    total = _N_WORKERS * b_per_w
    seq = ids.shape[1]
    per_row = seq // b_per_w
    mesh = plsc.VectorSubcoreMesh(core_axis_name="c", subcore_axis_name="s")

    @functools.partial(
        pl.kernel,
        mesh=mesh,
        out_type=jax.ShapeDtypeStruct((total, _FACT), jnp.float32),
        scratch_types=[
            pltpu.VMEM((b_per_w,), jnp.int32),
            pltpu.VMEM((b_per_w, _FACT), jnp.float32),
            pltpu.SemaphoreType.DMA,
            pltpu.SemaphoreType.DMA,
        ],
    )
    def gather_kernel(table_hbm, ids_hbm, out_hbm, idx_v, rows_v, gsem, osem):
        wid = lax.axis_index("s") * nc + lax.axis_index("c")
        row = wid // per_row
        col0 = (wid % per_row) * b_per_w
        pltpu.sync_copy(ids_hbm.at[row, pl.ds(col0, b_per_w)], idx_v)
        gathers = []
        for j in range(n_chunks):
            gathers.append(
                pltpu.async_copy(
                    table_hbm.at[idx_v.at[pl.ds(j * _CHUNK, _CHUNK)]],
                    rows_v.at[pl.ds(j * _CHUNK, _CHUNK)],
                    gsem,
                )
            )
        base = wid * b_per_w
        outs = []
        for j in range(n_chunks):
            gathers[j].wait()
            outs.append(
                pltpu.async_copy(
                    rows_v.at[pl.ds(j * _CHUNK, _CHUNK)],
                    out_hbm.at[pl.ds(base + j * _CHUNK, _CHUNK)],
                    osem,
                )
            )
        for o in outs:
            o.wait()

    return gather_kernel(table, ids)


def _tc_project(x, w, m_blk):
    m = x.shape[0]

    def mm(x_ref, w_ref, o_ref):
        o_ref[...] = lax.dot_general(
            x_ref[...].astype(jnp.bfloat16),
            w_ref[...],
            (((1,), (1,)), ((), ())),
            preferred_element_type=jnp.float32,
        )

    return pl.pallas_call(
        mm,
        grid=(m // m_blk,),
        in_specs=[
            pl.BlockSpec((m_blk, _FACT), lambda i: (i, 0)),
            pl.BlockSpec((_HIDDEN, _FACT), lambda i: (0, 0)),
        ],
        out_specs=pl.BlockSpec((m_blk, _HIDDEN), lambda i: (i, 0)),
        out_shape=jax.ShapeDtypeStruct((m, _HIDDEN), jnp.float32),
        compiler_params=pltpu.CompilerParams(
            dimension_semantics=("parallel",),
        ),
    )(x, w.astype(jnp.bfloat16))


def kernel(input_ids, embed_weight, proj_weight):
    b, s = input_ids.shape
    total = b * s
    n_chunks = total // (_N_WORKERS * _CHUNK)
    x = _sc_gather(embed_weight, input_ids.astype(jnp.int32), n_chunks)
    y = _tc_project(x, proj_weight, m_blk=1024)
    return y.reshape(b, s, _HIDDEN)
